# Initial kernel scaffold; baseline (speedup 1.0000x reference)
import jax
import jax.numpy as jnp
from jax.experimental import pallas as pl


def kernel(x, pos, edge_attr, edge_index, batch, batch_size, ne_w1, ne_b1, ne_gn_g, ne_gn_b, ne_w2, ne_b2, pe_w1, pe_b1, pe_gn_g, pe_gn_b, pe_w2, pe_b2, ee_w1, ee_b1, ee_gn_g, ee_gn_b, ee_w2, ee_b2, conv_t, conv_w1, conv_b1, conv_lng, conv_lnb, conv_w2, conv_b2, ln_g, ln_b, dec_w, dec_b):
    raise NotImplementedError("write your pallas kernel here")


# speedup vs baseline: 1.9785x; 1.9785x over previous
import functools

import jax
import jax.numpy as jnp
from jax.experimental import pallas as pl
from jax.experimental.pallas import tpu as pltpu

_INTERPRET = False

EPS = 1e-7
_NUM_GRAPHS = 64


def _row_blocks(n):
    for cand in (10000, 12800, 8192, 4096, 2048, 1024, 512, 256, 128, 64, 32, 16, 8):
        if n % cand == 0:
            return cand
    return n


def _group_mats(c, num_groups):
    g = c // num_groups
    col = jax.lax.broadcasted_iota(jnp.int32, (c, num_groups), 0) // g
    row = jax.lax.broadcasted_iota(jnp.int32, (c, num_groups), 1)
    m_avg = jnp.where(col == row, 1.0 / g, 0.0).astype(jnp.float32)
    m_bcast = jnp.where(col == row, 1.0, 0.0).astype(jnp.float32).T
    return m_avg, m_bcast


def _gn(x, num_groups, gamma, beta):
    c = x.shape[-1]
    m_avg, m_bcast = _group_mats(c, num_groups)
    mu_g = jnp.dot(x, m_avg, preferred_element_type=jnp.float32)
    ex2_g = jnp.dot(x * x, m_avg, preferred_element_type=jnp.float32)
    var_g = ex2_g - mu_g * mu_g
    mu = jnp.dot(mu_g, m_bcast, preferred_element_type=jnp.float32)
    inv = jnp.dot(jax.lax.rsqrt(var_g + 1e-5), m_bcast,
                  preferred_element_type=jnp.float32)
    return (x - mu) * inv * gamma + beta


def _enc_body(x_ref, pos_ref, nw1, nb1, ng, nbt, nw2, nb2,
              pw1, pb1, pg, pbt, pw2, pb2, o_ref):
    h = jnp.dot(x_ref[...], nw1[...], preferred_element_type=jnp.float32) + nb1[...]
    h = _gn(h, 1, ng[...], nbt[...])
    h = jnp.maximum(h, 0.0)
    h = jnp.dot(h, nw2[...], preferred_element_type=jnp.float32) + nb2[...]
    q = jnp.dot(pos_ref[...], pw1[...], preferred_element_type=jnp.float32) + pb1[...]
    q = _gn(q, 4, pg[...], pbt[...])
    q = jnp.maximum(q, 0.0)
    q = jnp.dot(q, pw2[...], preferred_element_type=jnp.float32) + pb2[...]
    o_ref[...] = jnp.concatenate([h, q], axis=1)


def _edge_enc_body(ea_ref, w1, b1, g, bt, w2, b2, o_ref):
    e = jnp.dot(ea_ref[...], w1[...], preferred_element_type=jnp.float32) + b1[...]
    e = _gn(e, 4, g[...], bt[...])
    e = jnp.maximum(e, 0.0)
    o_ref[...] = jnp.dot(e, w2[...], preferred_element_type=jnp.float32) + b2[...]


def _ln(x, gamma, beta):
    mu = jnp.mean(x, axis=-1, keepdims=True)
    var = jnp.mean((x - mu) ** 2, axis=-1, keepdims=True)
    return (x - mu) * jax.lax.rsqrt(var + 1e-5) * gamma + beta


def _conv_mlp_body(xin_ref, num_ref, den_ref, base_ref,
                   w1, b1, lng, lnb, w2, b2, o_ref):
    agg = num_ref[...] / den_ref[...]
    out = agg + xin_ref[...]
    z = jnp.dot(out, w1[...], preferred_element_type=jnp.float32) + b1[...]
    z = _ln(z, lng[...], lnb[...])
    z = jnp.maximum(z, 0.0)
    z = jnp.dot(z, w2[...], preferred_element_type=jnp.float32) + b2[...]
    o_ref[...] = base_ref[...] + z


def _lnrelu_body(h_ref, g, b, o_ref):
    o_ref[...] = jnp.maximum(_ln(h_ref[...], g[...], b[...]), 0.0)


def _decode_body(h_ref, g, b, dw, db, o_ref):
    z = jnp.maximum(_ln(h_ref[...], g[...], b[...]), 0.0)
    o_ref[...] = jnp.dot(z, dw[...], preferred_element_type=jnp.float32) + db[...]


def _full_spec(*shape):
    return pl.BlockSpec(shape, lambda i: tuple(0 for _ in shape))


def _call_rowblocked(body, n, out_dim, blocked_args, full_args):
    bk = _row_blocks(n)
    grid = n // bk
    in_specs = [pl.BlockSpec((bk, a.shape[1]), lambda i: (i, 0))
                for a in blocked_args]
    in_specs += [_full_spec(*a.shape) for a in full_args]
    return pl.pallas_call(
        body,
        grid=(grid,),
        in_specs=in_specs,
        out_specs=pl.BlockSpec((bk, out_dim), lambda i: (i, 0)),
        out_shape=jax.ShapeDtypeStruct((n, out_dim), jnp.float32),
        interpret=_INTERPRET,
    )(*blocked_args, *full_args)


def _r2(v):
    return v.reshape(1, -1)


def kernel(x, pos, edge_attr, edge_index, batch, batch_size,
           ne_w1, ne_b1, ne_gn_g, ne_gn_b, ne_w2, ne_b2,
           pe_w1, pe_b1, pe_gn_g, pe_gn_b, pe_w2, pe_b2,
           ee_w1, ee_b1, ee_gn_g, ee_gn_b, ee_w2, ee_b2,
           conv_t, conv_w1, conv_b1, conv_lng, conv_lnb, conv_w2, conv_b2,
           ln_g, ln_b, dec_w, dec_b):
    n = x.shape[0]
    e_cnt = edge_attr.shape[0]
    c = ee_w2.shape[1]
    num_layers = conv_w1.shape[0]
    src = edge_index[0]
    dst = edge_index[1]

    h = _call_rowblocked(
        _enc_body, n, c, [x, pos],
        [ne_w1, _r2(ne_b1), _r2(ne_gn_g), _r2(ne_gn_b), ne_w2, _r2(ne_b2),
         pe_w1, _r2(pe_b1), _r2(pe_gn_g), _r2(pe_gn_b), pe_w2, _r2(pe_b2)])

    e = _call_rowblocked(
        _edge_enc_body, e_cnt, c, [edge_attr],
        [ee_w1, _r2(ee_b1), _r2(ee_gn_g), _r2(ee_gn_b), ee_w2, _r2(ee_b2)])

    def edge_phase(hin, t):
        msg = jnp.maximum(hin[src] + e, 0.0) + EPS
        ex = jnp.exp(msg * t)
        den = jax.ops.segment_sum(ex, dst, num_segments=n) + 1e-16
        num = jax.ops.segment_sum(msg * ex, dst, num_segments=n)
        return num, den

    def conv_mlp(xin, num, den, base, i):
        return _call_rowblocked(
            _conv_mlp_body, n, c, [xin, num, den, base],
            [conv_w1[i], _r2(conv_b1[i]), _r2(conv_lng[i]), _r2(conv_lnb[i]),
             conv_w2[i], _r2(conv_b2[i])])

    num, den = edge_phase(h, conv_t[0])
    h = conv_mlp(h, num, den, jnp.zeros_like(h), 0)
    for i in range(1, num_layers):
        r = _call_rowblocked(_lnrelu_body, n, c, [h],
                             [_r2(ln_g[i]), _r2(ln_b[i])])
        num, den = edge_phase(r, conv_t[i])
        h = conv_mlp(r, num, den, h, i)

    dec = _call_rowblocked(
        _decode_body, n, dec_w.shape[1], [h],
        [_r2(ln_g[0]), _r2(ln_b[0]), dec_w, _r2(dec_b)])
    out = jax.ops.segment_max(dec, batch, num_segments=_NUM_GRAPHS)
    return out


# speedup vs baseline: 3.1840x; 1.6093x over previous
import functools

import jax
import jax.numpy as jnp
from jax import lax
from jax.experimental import pallas as pl
from jax.experimental.pallas import tpu as pltpu
from jax.experimental.pallas import tpu_sc as plsc

_INTERPRET = False

EPS = 1e-7
_NUM_GRAPHS = 64

_HALF = 50000
_ACC_ROWS = 50016
_ZROWS = 3126
_DROWS = 3120
_DTAIL = _HALF - 16 * _DROWS
_K = 256
_LANES = 16


def _row_blocks(n):
    for cand in (10000, 12800, 8192, 4096, 2048, 1024, 512, 256, 128, 64, 32, 16, 8):
        if n % cand == 0:
            return cand
    return n


def _group_mats(c, num_groups):
    g = c // num_groups
    col = jax.lax.broadcasted_iota(jnp.int32, (c, num_groups), 0) // g
    row = jax.lax.broadcasted_iota(jnp.int32, (c, num_groups), 1)
    m_avg = jnp.where(col == row, 1.0 / g, 0.0).astype(jnp.float32)
    m_bcast = jnp.where(col == row, 1.0, 0.0).astype(jnp.float32).T
    return m_avg, m_bcast


def _gn(x, num_groups, gamma, beta):
    c = x.shape[-1]
    m_avg, m_bcast = _group_mats(c, num_groups)
    mu_g = jnp.dot(x, m_avg, preferred_element_type=jnp.float32)
    ex2_g = jnp.dot(x * x, m_avg, preferred_element_type=jnp.float32)
    var_g = ex2_g - mu_g * mu_g
    mu = jnp.dot(mu_g, m_bcast, preferred_element_type=jnp.float32)
    inv = jnp.dot(jax.lax.rsqrt(var_g + 1e-5), m_bcast,
                  preferred_element_type=jnp.float32)
    return (x - mu) * inv * gamma + beta


def _enc_body(x_ref, pos_ref, nw1, nb1, ng, nbt, nw2, nb2,
              pw1, pb1, pg, pbt, pw2, pb2, o_ref):
    h = jnp.dot(x_ref[...], nw1[...], preferred_element_type=jnp.float32) + nb1[...]
    h = _gn(h, 1, ng[...], nbt[...])
    h = jnp.maximum(h, 0.0)
    h = jnp.dot(h, nw2[...], preferred_element_type=jnp.float32) + nb2[...]
    q = jnp.dot(pos_ref[...], pw1[...], preferred_element_type=jnp.float32) + pb1[...]
    q = _gn(q, 4, pg[...], pbt[...])
    q = jnp.maximum(q, 0.0)
    q = jnp.dot(q, pw2[...], preferred_element_type=jnp.float32) + pb2[...]
    o_ref[...] = jnp.concatenate([h, q], axis=1)


def _edge_enc_body(ea_ref, w1, b1, g, bt, w2, b2, o_ref):
    e = jnp.dot(ea_ref[...], w1[...], preferred_element_type=jnp.float32) + b1[...]
    e = _gn(e, 4, g[...], bt[...])
    e = jnp.maximum(e, 0.0)
    o_ref[...] = jnp.dot(e, w2[...], preferred_element_type=jnp.float32) + b2[...]


def _ln(x, gamma, beta):
    mu = jnp.mean(x, axis=-1, keepdims=True)
    var = jnp.mean((x - mu) ** 2, axis=-1, keepdims=True)
    return (x - mu) * jax.lax.rsqrt(var + 1e-5) * gamma + beta


def _conv_mlp_body(xin_ref, nd0_ref, nd1_ref, base_ref,
                   w1, b1, lng, lnb, w2, b2, o_ref):
    nd0 = nd0_ref[...]
    nd1 = nd1_ref[...]
    agg = jnp.concatenate(
        [nd0[:, :16] / (nd0[:, 16:] + 1e-16),
         nd1[:, :16] / (nd1[:, 16:] + 1e-16)], axis=1)
    out = agg + xin_ref[...]
    z = jnp.dot(out, w1[...], preferred_element_type=jnp.float32) + b1[...]
    z = _ln(z, lng[...], lnb[...])
    z = jnp.maximum(z, 0.0)
    z = jnp.dot(z, w2[...], preferred_element_type=jnp.float32) + b2[...]
    o_ref[...] = base_ref[...] + z


def _lnrelu_body(h_ref, g, b, o_ref):
    o_ref[...] = jnp.maximum(_ln(h_ref[...], g[...], b[...]), 0.0)


def _decode_body(h_ref, g, b, dw, db, o_ref):
    z = jnp.maximum(_ln(h_ref[...], g[...], b[...]), 0.0)
    o_ref[...] = jnp.dot(z, dw[...], preferred_element_type=jnp.float32) + db[...]


def _full_spec(*shape):
    return pl.BlockSpec(shape, lambda i: tuple(0 for _ in shape))


def _call_rowblocked(body, n, out_dim, blocked_args, full_args):
    bk = _row_blocks(n)
    grid = n // bk
    in_specs = [pl.BlockSpec((bk, a.shape[1]), lambda i: (i, 0))
                for a in blocked_args]
    in_specs += [_full_spec(*a.shape) for a in full_args]
    return pl.pallas_call(
        body,
        grid=(grid,),
        in_specs=in_specs,
        out_specs=pl.BlockSpec((bk, out_dim), lambda i: (i, 0)),
        out_shape=jax.ShapeDtypeStruct((n, out_dim), jnp.float32),
        interpret=_INTERPRET,
    )(*blocked_args, *full_args)


def _r2(v):
    return v.reshape(1, -1)


def _split_halves(a):
    return jnp.concatenate([a[:, :16], a[:, 16:]], axis=0)


def _sc_edge_pass(hs, es, src, dst, t_arr, zeros_h):
    nn = hs.shape[0] // 2
    ee = src.shape[0]
    nchunks = ee // _K
    chunks_per_tile = (nchunks + 15) // 16
    mesh = plsc.VectorSubcoreMesh(core_axis_name="c", subcore_axis_name="s")

    @functools.partial(
        pl.kernel,
        mesh=mesh,
        compiler_params=pltpu.CompilerParams(use_tc_tiling_on_sc=False,
                                             internal_scratch_in_bytes=0),
        out_type=jax.ShapeDtypeStruct((2 * nn, 32), jnp.float32),
        scratch_types=[
            pltpu.VMEM((_K,), jnp.int32),
            pltpu.VMEM((_K,), jnp.int32),
            pltpu.VMEM((_K,), jnp.int32),
            pltpu.VMEM((_K // 128, 128), jnp.int32),
            pltpu.VMEM((_K, 16), jnp.float32),
            pltpu.VMEM((_K, 16), jnp.float32),
            pltpu.VMEM((_K, 32), jnp.float32),
            pltpu.VMEM((16,), jnp.float32),
            pltpu.VMEM_SHARED((_ACC_ROWS, 32), jnp.float32),
            pltpu.SemaphoreType.DMA,
        ],
    )
    def k(hs_h, es_h, src_h, dst_h, t_h, z_h, out_h,
          srcv, dstv, gixv, sixv, hv, ev, valv, tv, acc, sem):
        c = lax.axis_index("c")
        s = lax.axis_index("s")
        pltpu.sync_copy(t_h, tv)

        for r in (0, 1):
            base_node = r * _HALF
            pltpu.sync_copy(z_h, acc.at[pl.ds(s * _ZROWS, _ZROWS)])
            plsc.subcore_barrier()

            def chunk_body(ci, carry):
                g = s + ci * 16

                @pl.when(g < nchunks)
                def _():
                    b = g * _K
                    pltpu.sync_copy(src_h.at[pl.ds(b, _K)], srcv)
                    pltpu.sync_copy(dst_h.at[pl.ds(b, _K)], dstv)
                    pltpu.sync_copy(es_h.at[pl.ds(c * ee + b, _K)], ev)
                    for j in range(_K // _LANES):
                        gixv[pl.ds(j * _LANES, _LANES)] = (
                            srcv[pl.ds(j * _LANES, _LANES)] + c * nn)
                    pltpu.async_copy(hs_h.at[gixv], hv, sem).wait()
                    for j in range(_K // _LANES):
                        dl = dstv[pl.ds(j * _LANES, _LANES)] - base_node
                        ok = (dl >= 0) & (dl < _HALF)
                        six = jnp.where(ok, dl, _HALF)
                        sixv[j // 8, pl.ds((j % 8) * _LANES, _LANES)] = six

                    def vbody(gi, vc):
                        m = jnp.maximum(
                            hv[gi, pl.ds(0, 16)] + ev[gi, pl.ds(0, 16)],
                            0.0) + EPS
                        x = jnp.exp(m * tv[...])
                        valv[gi, pl.ds(0, 16)] = m * x
                        valv[gi, pl.ds(16, 16)] = x
                        return vc

                    lax.fori_loop(0, _K, vbody, 0)
                    for i in range(_K // 128):
                        pltpu.sync_copy(valv.at[pl.ds(i * 128, 128)],
                                        acc.at[sixv.at[i]], add=True)

                return carry

            lax.fori_loop(0, chunks_per_tile, chunk_body, 0)
            plsc.subcore_barrier()
            pltpu.sync_copy(
                acc.at[pl.ds(s * _DROWS, _DROWS)],
                out_h.at[pl.ds(c * nn + base_node + s * _DROWS, _DROWS)])

            @pl.when(s == 0)
            def _():
                pltpu.sync_copy(
                    acc.at[pl.ds(16 * _DROWS, _DTAIL)],
                    out_h.at[pl.ds(c * nn + base_node + 16 * _DROWS, _DTAIL)])
            plsc.subcore_barrier()

    return k(hs, es, src, dst, t_arr, zeros_h)


def kernel(x, pos, edge_attr, edge_index, batch, batch_size,
           ne_w1, ne_b1, ne_gn_g, ne_gn_b, ne_w2, ne_b2,
           pe_w1, pe_b1, pe_gn_g, pe_gn_b, pe_w2, pe_b2,
           ee_w1, ee_b1, ee_gn_g, ee_gn_b, ee_w2, ee_b2,
           conv_t, conv_w1, conv_b1, conv_lng, conv_lnb, conv_w2, conv_b2,
           ln_g, ln_b, dec_w, dec_b):
    n = x.shape[0]
    e_cnt = edge_attr.shape[0]
    c = ee_w2.shape[1]
    num_layers = conv_w1.shape[0]
    src = edge_index[0]
    dst = edge_index[1]

    h = _call_rowblocked(
        _enc_body, n, c, [x, pos],
        [ne_w1, _r2(ne_b1), _r2(ne_gn_g), _r2(ne_gn_b), ne_w2, _r2(ne_b2),
         pe_w1, _r2(pe_b1), _r2(pe_gn_g), _r2(pe_gn_b), pe_w2, _r2(pe_b2)])

    e = _call_rowblocked(
        _edge_enc_body, e_cnt, c, [edge_attr],
        [ee_w1, _r2(ee_b1), _r2(ee_gn_g), _r2(ee_gn_b), ee_w2, _r2(ee_b2)])

    es = _split_halves(e)
    zeros_h = jnp.zeros((_ZROWS, 32), jnp.float32)

    def edge_phase(hin, t):
        nd = _sc_edge_pass(_split_halves(hin), es, src, dst,
                           jnp.full((16,), t, jnp.float32), zeros_h)
        return nd[:n], nd[n:]

    def conv_mlp(xin, nd0, nd1, base, i):
        return _call_rowblocked(
            _conv_mlp_body, n, c, [xin, nd0, nd1, base],
            [conv_w1[i], _r2(conv_b1[i]), _r2(conv_lng[i]), _r2(conv_lnb[i]),
             conv_w2[i], _r2(conv_b2[i])])

    nd0, nd1 = edge_phase(h, conv_t[0])
    h = conv_mlp(h, nd0, nd1, jnp.zeros_like(h), 0)
    for i in range(1, num_layers):
        r = _call_rowblocked(_lnrelu_body, n, c, [h],
                             [_r2(ln_g[i]), _r2(ln_b[i])])
        nd0, nd1 = edge_phase(r, conv_t[i])
        h = conv_mlp(r, nd0, nd1, h, i)

    dec = _call_rowblocked(
        _decode_body, n, dec_w.shape[1], [h],
        [_r2(ln_g[0]), _r2(ln_b[0]), dec_w, _r2(dec_b)])
    out = jax.ops.segment_max(dec, batch, num_segments=_NUM_GRAPHS)
    return out
